# baseline (speedup 1.0000x reference)
import functools

import jax
import jax.numpy as jnp
from jax.experimental import pallas as pl
from jax.experimental.pallas import tpu as pltpu

_ATOM_PAD = 128
_HID = 512
_NB = 16
_TB = 256


def _conv_stack_kernel(xs_ref, w1_ref, b1_ref, w2_ref, b2_ref, w3_ref, b3_ref,
                       feat_ref, p1_ref, p2_ref, p3_ref):
    nb = _NB
    xs = xs_ref[...]
    rows = {
        0: xs[:, 0:20].reshape(nb, 10, 2, 2, 16, 128)[:, :, 0],
        1: xs[:, 1:21].reshape(nb, 10, 2, 2, 16, 128)[:, :, 0],
        2: xs[:, 1:21].reshape(nb, 10, 2, 2, 16, 128)[:, :, 1],
    }
    for alpha in range(2):
        for a in range(2):
            src = rows[alpha + a]
            for beta in range(2):
                p1_ref[:, alpha, :, beta, :, a * 128:(a + 1) * 128] = \
                    src[:, :, beta, :, :]
    y1 = jnp.dot(p1_ref[...].reshape(nb * 640, 256), w1_ref[...],
                 preferred_element_type=jnp.float32)
    y1 = jnp.maximum(y1 + b1_ref[...], 0.0).astype(jnp.bfloat16)
    c1 = y1.reshape(nb, 2, 10, 2, 16, 128)

    for i in range(4):
        for j in range(4):
            t = i * 4 + j
            val = c1[:, i & 1, (i >> 1):(i >> 1) + 9,
                     j & 1, (j >> 1):(j >> 1) + 9, :32]
            p2_ref[:, :, 0:9, t * 32:(t + 1) * 32] = val
    y2 = jnp.dot(p2_ref[...].reshape(nb * 144, 512), w2_ref[...],
                 preferred_element_type=jnp.float32)
    y2 = jnp.maximum(y2 + b2_ref[...], 0.0).astype(jnp.bfloat16)
    c2 = y2.reshape(nb, 9, 16, 128)

    for i in range(3):
        for j in range(3):
            t = i * 3 + j
            p3_ref[:, :, 0:7, t * 64:(t + 1) * 64] = c2[:, i:i + 7, j:j + 7, :64]
    y3 = jnp.dot(p3_ref[...].reshape(nb * 112, 576), w3_ref[...],
                 preferred_element_type=jnp.float32)
    y3 = jnp.maximum(y3 + b3_ref[...], 0.0).astype(jnp.bfloat16)
    c3 = y3.reshape(nb, 7, 16, 128)

    for p in range(7):
        for q in range(7):
            feat_ref[:, (p * 7 + q) * 64:(p * 7 + q + 1) * 64] = c3[:, p, q, :64]


def _head_kernel(f_ref, w1_ref, b1_ref, wv2_ref, bv2_ref, wa2_ref, ba2_ref,
                 mask_ref, sup_ref, q_ref, dist_ref, *, n_actions):
    h = jnp.dot(f_ref[...], w1_ref[...], preferred_element_type=jnp.float32)
    h = jnp.maximum(h + b1_ref[...], 0.0)
    hv = h[:, :_HID]
    ha = h[:, _HID:]

    value = jnp.dot(hv, wv2_ref[...], preferred_element_type=jnp.float32) + bv2_ref[...]
    adv = jnp.dot(ha, wa2_ref[...], preferred_element_type=jnp.float32) + ba2_ref[...]

    adv_mean = adv[:, :_ATOM_PAD]
    for a in range(1, n_actions):
        adv_mean = adv_mean + adv[:, a * _ATOM_PAD:(a + 1) * _ATOM_PAD]
    adv_mean = adv_mean * (1.0 / n_actions)

    base = value - adv_mean + mask_ref[...]
    for a in range(n_actions):
        qa = base + adv[:, a * _ATOM_PAD:(a + 1) * _ATOM_PAD]
        m = jnp.max(qa, axis=-1, keepdims=True)
        e = jnp.exp(qa - m)
        s = jnp.sum(e, axis=-1, keepdims=True)
        inv = pl.reciprocal(s, approx=True)
        dist_ref[:, a * _ATOM_PAD:(a + 1) * _ATOM_PAD] = jnp.maximum(e * inv, 0.001)

    q_ref[...] = jnp.dot(dist_ref[...], sup_ref[...],
                         preferred_element_type=jnp.float32)


def _permute_conv1_rows(c1w):
    return (c1w.reshape(2, 4, 2, 4, 4, 128)
            .transpose(0, 2, 1, 3, 4, 5)
            .reshape(256, 128))


def _host_s2d(x):
    B = x.shape[0]
    xr = x.reshape(B, 4, 21, 4, 21, 4).transpose(0, 2, 4, 3, 5, 1)
    xs = xr.reshape(B, 21, 21, 64).astype(jnp.bfloat16)
    xsp = jnp.pad(xs, ((0, 0), (0, 0), (0, 3), (0, 0)))
    s0 = xsp[:, :, 0:22].reshape(B, 21, 11, 2, 64).transpose(0, 1, 3, 2, 4)
    s1 = xsp[:, :, 1:23].reshape(B, 21, 11, 2, 64).transpose(0, 1, 3, 2, 4)
    cat = jnp.concatenate([s0, s1], axis=-1)
    return jnp.pad(cat, ((0, 0), (0, 0), (0, 0), (0, 5), (0, 0)))


def kernel(x, c1w, c1b, c2w, c2b, c3w, c3b, w1h, b1h, wv2p, bv2p, wa2p, ba2p,
           mask, S):
    n_actions = 18
    B = x.shape[0]

    xs = _host_s2d(x)
    w1p = _permute_conv1_rows(c1w)

    nsteps = B // _NB
    feature = pl.pallas_call(
        _conv_stack_kernel,
        out_shape=jax.ShapeDtypeStruct((B, 3136), jnp.bfloat16),
        grid=(nsteps,),
        in_specs=[
            pl.BlockSpec((_NB, 21, 2, 16, 128), lambda i: (i, 0, 0, 0, 0)),
            pl.BlockSpec((256, 128), lambda i: (0, 0)),
            pl.BlockSpec((1, 128), lambda i: (0, 0)),
            pl.BlockSpec((512, 128), lambda i: (0, 0)),
            pl.BlockSpec((1, 128), lambda i: (0, 0)),
            pl.BlockSpec((576, 128), lambda i: (0, 0)),
            pl.BlockSpec((1, 128), lambda i: (0, 0)),
        ],
        out_specs=pl.BlockSpec((_NB, 3136), lambda i: (i, 0)),
        scratch_shapes=[
            pltpu.VMEM((_NB, 2, 10, 2, 16, 256), jnp.bfloat16),
            pltpu.VMEM((_NB, 9, 16, 512), jnp.bfloat16),
            pltpu.VMEM((_NB, 7, 16, 576), jnp.bfloat16),
        ],
        compiler_params=pltpu.CompilerParams(
            dimension_semantics=("parallel",),
            vmem_limit_bytes=100 * 1024 * 1024,
        ),
    )(xs, w1p, c1b.reshape(1, 128).astype(jnp.float32),
      c2w, c2b.reshape(1, 128).astype(jnp.float32),
      c3w, c3b.reshape(1, 128).astype(jnp.float32))

    tb = min(_TB, B)
    q = pl.pallas_call(
        functools.partial(_head_kernel, n_actions=n_actions),
        out_shape=jax.ShapeDtypeStruct((B, _ATOM_PAD), jnp.float32),
        grid=(B // tb,),
        in_specs=[
            pl.BlockSpec((tb, 3136), lambda i: (i, 0)),
            pl.BlockSpec((3136, 2 * _HID), lambda i: (0, 0)),
            pl.BlockSpec((1, 2 * _HID), lambda i: (0, 0)),
            pl.BlockSpec((_HID, _ATOM_PAD), lambda i: (0, 0)),
            pl.BlockSpec((1, _ATOM_PAD), lambda i: (0, 0)),
            pl.BlockSpec((_HID, n_actions * _ATOM_PAD), lambda i: (0, 0)),
            pl.BlockSpec((1, n_actions * _ATOM_PAD), lambda i: (0, 0)),
            pl.BlockSpec((1, _ATOM_PAD), lambda i: (0, 0)),
            pl.BlockSpec((n_actions * _ATOM_PAD, _ATOM_PAD), lambda i: (0, 0)),
        ],
        out_specs=pl.BlockSpec((tb, _ATOM_PAD), lambda i: (i, 0)),
        scratch_shapes=[pltpu.VMEM((tb, n_actions * _ATOM_PAD), jnp.float32)],
        compiler_params=pltpu.CompilerParams(
            dimension_semantics=("parallel",),
            vmem_limit_bytes=100 * 1024 * 1024,
        ),
    )(feature, w1h, b1h, wv2p, bv2p, wa2p, ba2p, mask, S)
    return q[:, :n_actions]


# speedup vs baseline: 1.0962x; 1.0962x over previous
import functools

import jax
import jax.numpy as jnp
from jax.experimental import pallas as pl
from jax.experimental.pallas import tpu as pltpu

_ATOM_PAD = 128
_HID = 512
_NB = 16
_TB = 256


def _conv_stack_kernel(xs_ref, w1_ref, b1_ref, w2_ref, b2_ref, w3_ref, b3_ref,
                       feat_ref, p1_ref, p2_ref, p3_ref):
    nb = _NB
    xs = xs_ref[...]
    rows = {
        0: xs[:, 0:20].reshape(nb, 10, 2, 2, 11, 64)[:, :, 0],
        1: xs[:, 1:21].reshape(nb, 10, 2, 2, 11, 64)[:, :, 0],
        2: xs[:, 1:21].reshape(nb, 10, 2, 2, 11, 64)[:, :, 1],
    }
    for alpha in range(2):
        for a in range(2):
            src = rows[alpha + a]
            for beta in range(2):
                for b2 in range(2):
                    s2 = beta + b2
                    lo = a * 128 + b2 * 64
                    if s2 < 2:
                        p1_ref[:, alpha, :, beta, 0:11, lo:lo + 64] = \
                            src[:, :, s2, :, :]
                    else:
                        p1_ref[:, alpha, :, beta, 0:10, lo:lo + 64] = \
                            src[:, :, 0, 1:11, :]
    y1 = jnp.dot(p1_ref[...].reshape(nb * 640, 256), w1_ref[...],
                 preferred_element_type=jnp.float32)
    y1 = jnp.maximum(y1 + b1_ref[...], 0.0).astype(jnp.bfloat16)
    c1 = y1.reshape(nb, 2, 10, 2, 16, 128)

    for i in range(4):
        for j in range(4):
            t = i * 4 + j
            val = c1[:, i & 1, (i >> 1):(i >> 1) + 9,
                     j & 1, (j >> 1):(j >> 1) + 9, :32]
            p2_ref[:, :, 0:9, t * 32:(t + 1) * 32] = val
    y2 = jnp.dot(p2_ref[...].reshape(nb * 144, 512), w2_ref[...],
                 preferred_element_type=jnp.float32)
    y2 = jnp.maximum(y2 + b2_ref[...], 0.0).astype(jnp.bfloat16)
    c2 = y2.reshape(nb, 9, 16, 128)

    for i in range(3):
        for j in range(3):
            t = i * 3 + j
            p3_ref[:, :, 0:7, t * 64:(t + 1) * 64] = c2[:, i:i + 7, j:j + 7, :64]
    y3 = jnp.dot(p3_ref[...].reshape(nb * 112, 576), w3_ref[...],
                 preferred_element_type=jnp.float32)
    y3 = jnp.maximum(y3 + b3_ref[...], 0.0).astype(jnp.bfloat16)
    c3 = y3.reshape(nb, 7, 16, 128)

    for p in range(7):
        for q in range(7):
            feat_ref[:, (p * 7 + q) * 64:(p * 7 + q + 1) * 64] = c3[:, p, q, :64]


def _head_kernel(f_ref, w1_ref, b1_ref, wv2_ref, bv2_ref, wa2_ref, ba2_ref,
                 mask_ref, sup_ref, q_ref, dist_ref, *, n_actions):
    h = jnp.dot(f_ref[...], w1_ref[...], preferred_element_type=jnp.float32)
    h = jnp.maximum(h + b1_ref[...], 0.0)
    hv = h[:, :_HID]
    ha = h[:, _HID:]

    value = jnp.dot(hv, wv2_ref[...], preferred_element_type=jnp.float32) + bv2_ref[...]
    adv = jnp.dot(ha, wa2_ref[...], preferred_element_type=jnp.float32) + ba2_ref[...]

    adv_mean = adv[:, :_ATOM_PAD]
    for a in range(1, n_actions):
        adv_mean = adv_mean + adv[:, a * _ATOM_PAD:(a + 1) * _ATOM_PAD]
    adv_mean = adv_mean * (1.0 / n_actions)

    base = value - adv_mean + mask_ref[...]
    for a in range(n_actions):
        qa = base + adv[:, a * _ATOM_PAD:(a + 1) * _ATOM_PAD]
        m = jnp.max(qa, axis=-1, keepdims=True)
        e = jnp.exp(qa - m)
        s = jnp.sum(e, axis=-1, keepdims=True)
        inv = pl.reciprocal(s, approx=True)
        dist_ref[:, a * _ATOM_PAD:(a + 1) * _ATOM_PAD] = jnp.maximum(e * inv, 0.001)

    q_ref[...] = jnp.dot(dist_ref[...], sup_ref[...],
                         preferred_element_type=jnp.float32)


def _permute_conv1_rows(c1w):
    return (c1w.reshape(2, 4, 2, 4, 4, 128)
            .transpose(0, 2, 1, 3, 4, 5)
            .reshape(256, 128))


def _host_s2d(x):
    B = x.shape[0]
    xp = jnp.pad(x, ((0, 0), (0, 0), (0, 0), (0, 4)))
    xr = xp.reshape(B, 4, 21, 4, 11, 2, 4)
    return (xr.transpose(0, 2, 5, 4, 3, 6, 1)
            .reshape(B, 21, 2, 11, 64).astype(jnp.bfloat16))


def kernel(x, c1w, c1b, c2w, c2b, c3w, c3b, w1h, b1h, wv2p, bv2p, wa2p, ba2p,
           mask, S):
    n_actions = 18
    B = x.shape[0]

    xs = _host_s2d(x)
    w1p = _permute_conv1_rows(c1w)

    nsteps = B // _NB
    feature = pl.pallas_call(
        _conv_stack_kernel,
        out_shape=jax.ShapeDtypeStruct((B, 3136), jnp.bfloat16),
        grid=(nsteps,),
        in_specs=[
            pl.BlockSpec((_NB, 21, 2, 11, 64), lambda i: (i, 0, 0, 0, 0)),
            pl.BlockSpec((256, 128), lambda i: (0, 0)),
            pl.BlockSpec((1, 128), lambda i: (0, 0)),
            pl.BlockSpec((512, 128), lambda i: (0, 0)),
            pl.BlockSpec((1, 128), lambda i: (0, 0)),
            pl.BlockSpec((576, 128), lambda i: (0, 0)),
            pl.BlockSpec((1, 128), lambda i: (0, 0)),
        ],
        out_specs=pl.BlockSpec((_NB, 3136), lambda i: (i, 0)),
        scratch_shapes=[
            pltpu.VMEM((_NB, 2, 10, 2, 16, 256), jnp.bfloat16),
            pltpu.VMEM((_NB, 9, 16, 512), jnp.bfloat16),
            pltpu.VMEM((_NB, 7, 16, 576), jnp.bfloat16),
        ],
        compiler_params=pltpu.CompilerParams(
            dimension_semantics=("parallel",),
            vmem_limit_bytes=100 * 1024 * 1024,
        ),
    )(xs, w1p, c1b.reshape(1, 128).astype(jnp.float32),
      c2w, c2b.reshape(1, 128).astype(jnp.float32),
      c3w, c3b.reshape(1, 128).astype(jnp.float32))

    tb = min(_TB, B)
    q = pl.pallas_call(
        functools.partial(_head_kernel, n_actions=n_actions),
        out_shape=jax.ShapeDtypeStruct((B, _ATOM_PAD), jnp.float32),
        grid=(B // tb,),
        in_specs=[
            pl.BlockSpec((tb, 3136), lambda i: (i, 0)),
            pl.BlockSpec((3136, 2 * _HID), lambda i: (0, 0)),
            pl.BlockSpec((1, 2 * _HID), lambda i: (0, 0)),
            pl.BlockSpec((_HID, _ATOM_PAD), lambda i: (0, 0)),
            pl.BlockSpec((1, _ATOM_PAD), lambda i: (0, 0)),
            pl.BlockSpec((_HID, n_actions * _ATOM_PAD), lambda i: (0, 0)),
            pl.BlockSpec((1, n_actions * _ATOM_PAD), lambda i: (0, 0)),
            pl.BlockSpec((1, _ATOM_PAD), lambda i: (0, 0)),
            pl.BlockSpec((n_actions * _ATOM_PAD, _ATOM_PAD), lambda i: (0, 0)),
        ],
        out_specs=pl.BlockSpec((tb, _ATOM_PAD), lambda i: (i, 0)),
        scratch_shapes=[pltpu.VMEM((tb, n_actions * _ATOM_PAD), jnp.float32)],
        compiler_params=pltpu.CompilerParams(
            dimension_semantics=("parallel",),
            vmem_limit_bytes=100 * 1024 * 1024,
        ),
    )(feature, w1h, b1h, wv2p, bv2p, wa2p, ba2p, mask, S)
    return q[:, :n_actions]
